# baseline (speedup 1.0000x reference)
import functools

import jax
import jax.numpy as jnp
from jax import lax
from jax.experimental import pallas as pl
from jax.experimental.pallas import tpu as pltpu
from jax.experimental.pallas import tpu_sc as plsc

Q = 1024
D = 128
KB = 2048
NPAD = 100352
NBLK = NPAD // KB
GSZ = 128
NG = NPAD // GSZ
GB = KB // GSZ
QB1 = 512
QB2 = 128
TOPK = 10
R1 = (Q // 8) * NG * 8
BIGI = 2**30
NW = 32
QW = Q // NW
NR = QW * TOPK
INF = float("inf")


def _stage1(x_ref, ksq_ref, qsq_ref, kt_ref, dists_ref, gmin_ref):
    dot = lax.dot_general(x_ref[...], kt_ref[...],
                          (((1,), (1,)), ((), ())),
                          preferred_element_type=jnp.float32)
    dists = qsq_ref[...] - 2.0 * dot + ksq_ref[...]
    d3 = dists.reshape(QB1 // 8, 8, KB)
    for g in range(GB):
        dists_ref[:, g * 8:(g + 1) * 8, :] = d3[:, :, g * GSZ:(g + 1) * GSZ]
        gmin_ref[0, :, g:g + 1] = jnp.min(
            dists[:, g * GSZ:(g + 1) * GSZ], axis=1, keepdims=True)


def _stage2(gmin_ref, ridx_ref, kbase_ref):
    j = pl.program_id(0)
    c = gmin_ref[...]
    lane = lax.broadcasted_iota(jnp.int32, (QB2, NG), 1)
    qrow = lax.broadcasted_iota(jnp.int32, (QB2, 1), 0) + j * QB2
    rbase = (qrow // 8) * (NG * 8) + (qrow % 8)
    gs = []
    for _ in range(TOPK):
        m = jnp.min(c, axis=1, keepdims=True)
        pos = jnp.where(c == m, lane, BIGI)
        g = jnp.min(pos, axis=1, keepdims=True)
        gs.append(g)
        c = jnp.where(lane == g, jnp.inf, c)
    for r in range(TOPK):
        for i in range(r % 2, TOPK - 1, 2):
            a, b = gs[i], gs[i + 1]
            gs[i], gs[i + 1] = jnp.minimum(a, b), jnp.maximum(a, b)
    ridx_ref[...] = jnp.concatenate([rbase + g * 8 for g in gs], axis=1)
    kbase_ref[...] = jnp.concatenate([g * GSZ for g in gs], axis=1)


_GDN = lax.GatherDimensionNumbers(
    offset_dims=(), collapsed_slice_dims=(0,), start_index_map=(0,))


def _perm(v, idx):
    return lax.gather(v, idx[:, None], _GDN, (1,),
                      mode=lax.GatherScatterMode.PROMISE_IN_BOUNDS)


def _stage3(dists_ref, ridx_ref, kbase_ref, dpad_ref, ipad_ref,
            idx_v, kb_v, rows_v, dbuf, ibuf, sem):
    info = plsc.get_sparse_core_info()
    nc = info.num_cores
    wid = lax.axis_index("s") * nc + lax.axis_index("c")
    qlo = wid * QW

    pltpu.sync_copy(ridx_ref.at[pl.ds(qlo * TOPK, NR)], idx_v)
    pltpu.sync_copy(kbase_ref.at[pl.ds(qlo * TOPK, NR)], kb_v)
    copies = []
    for b in range(NR // 80):
        copies.append(pltpu.async_copy(
            dists_ref.at[idx_v.at[pl.ds(b * 80, 80)]],
            rows_v.at[pl.ds(b * 80, 80)], sem))
    for cp in copies:
        cp.wait()

    i16 = lax.broadcasted_iota(jnp.int32, (16,), 0)
    rots = [(i16 + sh) % 16 for sh in (8, 4, 2, 1)]
    lane_iotas = [i16 + u * 16 for u in range(8)]

    def mintree(v):
        for r in rots:
            v = jnp.minimum(v, _perm(v, r))
        return v

    def rowscan(row):
        v = rows_v[row, pl.ds(0, 16)]
        for u in range(1, 8):
            v = jnp.minimum(v, rows_v[row, pl.ds(u * 16, 16)])
        return mintree(v)

    def per_query(i, carry):
        base = i * TOPK
        rowmin = jnp.full((16,), INF, jnp.float32)
        for t in range(TOPK):
            rowmin = jnp.where(i16 == t, rowscan(base + t), rowmin)

        douts = jnp.full((16,), INF, jnp.float32)
        iouts = jnp.zeros((16,), jnp.int32)
        for t in range(TOPK):
            mvec = mintree(rowmin)
            rvec = mintree(jnp.where(rowmin == mvec, i16, 16))
            r = rvec[0]
            row = base + r
            pos = jnp.full((16,), 1024, jnp.int32)
            for u in range(8):
                vu = rows_v[row, pl.ds(u * 16, 16)]
                pos = jnp.minimum(
                    pos, jnp.where(vu == mvec, lane_iotas[u], 1024))
            p = mintree(pos)[0]
            kpos = i * TOPK + r
            kstart = (kpos // 16) * 16
            kvec = kb_v[pl.ds(kstart, 16)]
            kb = _perm(kvec, (i16 + (kpos - kstart)) % 16)[0]
            douts = jnp.where(i16 == t, mvec, douts)
            iouts = jnp.where(i16 == t, kb + p, iouts)
            u0 = (p // 16) * 16
            vold = rows_v[row, pl.ds(u0, 16)]
            vnew = jnp.where(i16 == p - u0, INF, vold)
            rows_v[row, pl.ds(u0, 16)] = vnew
            rowmin = jnp.where(i16 == rvec, rowscan(row), rowmin)
        dbuf[i, :] = douts
        ibuf[i, :] = iouts
        return carry

    lax.fori_loop(0, QW, per_query, 0)
    pltpu.sync_copy(dbuf, dpad_ref.at[pl.ds(qlo, QW)])
    pltpu.sync_copy(ibuf, ipad_ref.at[pl.ds(qlo, QW)])


def kernel(x, keys, k):
    n = keys.shape[0]
    q_sq = jnp.sum(x * x, axis=1, keepdims=True)
    k_sq = jnp.sum(keys * keys, axis=1)[None, :]
    k_sq = jnp.pad(k_sq, ((0, 0), (0, NPAD - n)), constant_values=1e30)
    keys_p = jnp.pad(keys, ((0, NPAD - n), (0, 0)))

    dists3, gmin3 = pl.pallas_call(
        _stage1,
        grid=(Q // QB1, NBLK),
        in_specs=[
            pl.BlockSpec((QB1, D), lambda q, j: (q, 0)),
            pl.BlockSpec((1, KB), lambda q, j: (0, j)),
            pl.BlockSpec((QB1, 1), lambda q, j: (q, 0)),
            pl.BlockSpec((KB, D), lambda q, j: (j, 0)),
        ],
        out_specs=[
            pl.BlockSpec((QB1 // 8, GB * 8, GSZ), lambda q, j: (q, j, 0)),
            pl.BlockSpec((1, QB1, GB), lambda q, j: (j, q, 0)),
        ],
        out_shape=[
            jax.ShapeDtypeStruct((Q // 8, NG * 8, GSZ), jnp.float32),
            jax.ShapeDtypeStruct((NBLK, Q, GB), jnp.float32),
        ],
        compiler_params=pltpu.CompilerParams(
            dimension_semantics=("parallel", "parallel"),
        ),
    )(x, k_sq, q_sq, keys_p)

    gmin = jnp.transpose(gmin3, (1, 0, 2)).reshape(Q, NG)

    ridx, kbase = pl.pallas_call(
        _stage2,
        grid=(Q // QB2,),
        in_specs=[pl.BlockSpec((QB2, NG), lambda j: (j, 0))],
        out_specs=[
            pl.BlockSpec((QB2, TOPK), lambda j: (j, 0)),
            pl.BlockSpec((QB2, TOPK), lambda j: (j, 0)),
        ],
        out_shape=[
            jax.ShapeDtypeStruct((Q, TOPK), jnp.int32),
            jax.ShapeDtypeStruct((Q, TOPK), jnp.int32),
        ],
        compiler_params=pltpu.CompilerParams(
            dimension_semantics=("arbitrary",),
        ),
    )(gmin)

    stage3 = functools.partial(
        pl.kernel,
        out_type=[
            jax.ShapeDtypeStruct((Q, 16), jnp.float32),
            jax.ShapeDtypeStruct((Q, 16), jnp.int32),
        ],
        mesh=plsc.VectorSubcoreMesh(core_axis_name="c", subcore_axis_name="s"),
        scratch_types=[
            pltpu.VMEM((NR,), jnp.int32),
            pltpu.VMEM((NR,), jnp.int32),
            pltpu.VMEM((NR, GSZ), jnp.float32),
            pltpu.VMEM((QW, 16), jnp.float32),
            pltpu.VMEM((QW, 16), jnp.int32),
            pltpu.SemaphoreType.DMA,
        ],
    )(_stage3)
    dpad, ipad = stage3(dists3.reshape(R1, GSZ),
                        ridx.reshape(Q * TOPK),
                        kbase.reshape(Q * TOPK))
    return (dpad[:, :TOPK], ipad[:, :TOPK])


# speedup vs baseline: 1.1409x; 1.1409x over previous
import functools

import jax
import jax.numpy as jnp
from jax import lax
from jax.experimental import pallas as pl
from jax.experimental.pallas import tpu as pltpu
from jax.experimental.pallas import tpu_sc as plsc

Q = 1024
D = 128
KB = 2048
NPAD = 100352
NBLK = NPAD // KB
GSZ = 128
NG = NPAD // GSZ
GB = KB // GSZ
QB1 = 1024
QB2 = 128
TOPK = 10
R1 = (Q // 8) * NG * 8
BIGI = 2**30
NW = 32
QW = Q // NW
NR = QW * TOPK
INF = float("inf")


def _stage1(x_ref, ksq_ref, qsq_ref, kt_ref, dists_ref, gmin_ref):
    dot = lax.dot_general(x_ref[...], kt_ref[...],
                          (((1,), (1,)), ((), ())),
                          preferred_element_type=jnp.float32)
    dists = qsq_ref[...] - 2.0 * dot + ksq_ref[...]
    d3 = dists.reshape(QB1 // 8, 8, KB)
    for g in range(GB):
        dists_ref[:, g * 8:(g + 1) * 8, :] = d3[:, :, g * GSZ:(g + 1) * GSZ]
        gmin_ref[0, :, g:g + 1] = jnp.min(
            dists[:, g * GSZ:(g + 1) * GSZ], axis=1, keepdims=True)


def _stage2(gmin_ref, ridx_ref, kbase_ref):
    j = pl.program_id(0)
    c = gmin_ref[...]
    lane = lax.broadcasted_iota(jnp.int32, (QB2, NG), 1)
    qrow = lax.broadcasted_iota(jnp.int32, (QB2, 1), 0) + j * QB2
    rbase = (qrow // 8) * (NG * 8) + (qrow % 8)
    gs = []
    for _ in range(TOPK):
        m = jnp.min(c, axis=1, keepdims=True)
        pos = jnp.where(c == m, lane, BIGI)
        g = jnp.min(pos, axis=1, keepdims=True)
        gs.append(g)
        c = jnp.where(lane == g, jnp.inf, c)
    for r in range(TOPK):
        for i in range(r % 2, TOPK - 1, 2):
            a, b = gs[i], gs[i + 1]
            gs[i], gs[i + 1] = jnp.minimum(a, b), jnp.maximum(a, b)
    ridx_ref[...] = jnp.concatenate([rbase + g * 8 for g in gs], axis=1)
    kbase_ref[...] = jnp.concatenate([g * GSZ for g in gs], axis=1)


_GDN = lax.GatherDimensionNumbers(
    offset_dims=(), collapsed_slice_dims=(0,), start_index_map=(0,))


def _perm(v, idx):
    return lax.gather(v, idx[:, None], _GDN, (1,),
                      mode=lax.GatherScatterMode.PROMISE_IN_BOUNDS)


def _stage3(dists_ref, ridx_ref, kbase_ref, dpad_ref, ipad_ref,
            idx_v, kb_v, rows_v, dbuf, ibuf, sem):
    info = plsc.get_sparse_core_info()
    nc = info.num_cores
    wid = lax.axis_index("s") * nc + lax.axis_index("c")
    qlo = wid * QW

    pltpu.sync_copy(ridx_ref.at[pl.ds(qlo * TOPK, NR)], idx_v)
    pltpu.sync_copy(kbase_ref.at[pl.ds(qlo * TOPK, NR)], kb_v)
    copies = []
    for b in range(NR // 80):
        copies.append(pltpu.async_copy(
            dists_ref.at[idx_v.at[pl.ds(b * 80, 80)]],
            rows_v.at[pl.ds(b * 80, 80)], sem))
    for cp in copies:
        cp.wait()

    i16 = lax.broadcasted_iota(jnp.int32, (16,), 0)
    rots = [(i16 + sh) % 16 for sh in (8, 4, 2, 1)]
    lane_iotas = [i16 + u * 16 for u in range(8)]

    def mintree(v):
        for r in rots:
            v = jnp.minimum(v, _perm(v, r))
        return v

    def rowscan(row):
        v = rows_v[row, pl.ds(0, 16)]
        for u in range(1, 8):
            v = jnp.minimum(v, rows_v[row, pl.ds(u * 16, 16)])
        return mintree(v)

    def per_query(i, carry):
        base = i * TOPK
        rowmin = jnp.full((16,), INF, jnp.float32)
        for t in range(TOPK):
            rowmin = jnp.where(i16 == t, rowscan(base + t), rowmin)

        douts = jnp.full((16,), INF, jnp.float32)
        iouts = jnp.zeros((16,), jnp.int32)
        for t in range(TOPK):
            mvec = mintree(rowmin)
            rvec = mintree(jnp.where(rowmin == mvec, i16, 16))
            r = rvec[0]
            row = base + r
            pos = jnp.full((16,), 1024, jnp.int32)
            for u in range(8):
                vu = rows_v[row, pl.ds(u * 16, 16)]
                pos = jnp.minimum(
                    pos, jnp.where(vu == mvec, lane_iotas[u], 1024))
            p = mintree(pos)[0]
            kpos = i * TOPK + r
            kstart = (kpos // 16) * 16
            kvec = kb_v[pl.ds(kstart, 16)]
            kb = _perm(kvec, (i16 + (kpos - kstart)) % 16)[0]
            douts = jnp.where(i16 == t, mvec, douts)
            iouts = jnp.where(i16 == t, kb + p, iouts)
            u0 = (p // 16) * 16
            vold = rows_v[row, pl.ds(u0, 16)]
            vnew = jnp.where(i16 == p - u0, INF, vold)
            rows_v[row, pl.ds(u0, 16)] = vnew
            rowmin = jnp.where(i16 == rvec, rowscan(row), rowmin)
        dbuf[i, :] = douts
        ibuf[i, :] = iouts
        return carry

    lax.fori_loop(0, QW, per_query, 0)
    pltpu.sync_copy(dbuf, dpad_ref.at[pl.ds(qlo, QW)])
    pltpu.sync_copy(ibuf, ipad_ref.at[pl.ds(qlo, QW)])


def kernel(x, keys, k):
    n = keys.shape[0]
    q_sq = jnp.sum(x * x, axis=1, keepdims=True)
    k_sq = jnp.sum(keys * keys, axis=1)[None, :]
    k_sq = jnp.pad(k_sq, ((0, 0), (0, NPAD - n)), constant_values=1e30)
    keys_p = jnp.pad(keys, ((0, NPAD - n), (0, 0)))

    dists3, gmin3 = pl.pallas_call(
        _stage1,
        grid=(Q // QB1, NBLK),
        in_specs=[
            pl.BlockSpec((QB1, D), lambda q, j: (q, 0)),
            pl.BlockSpec((1, KB), lambda q, j: (0, j)),
            pl.BlockSpec((QB1, 1), lambda q, j: (q, 0)),
            pl.BlockSpec((KB, D), lambda q, j: (j, 0)),
        ],
        out_specs=[
            pl.BlockSpec((QB1 // 8, GB * 8, GSZ), lambda q, j: (q, j, 0)),
            pl.BlockSpec((1, QB1, GB), lambda q, j: (j, q, 0)),
        ],
        out_shape=[
            jax.ShapeDtypeStruct((Q // 8, NG * 8, GSZ), jnp.float32),
            jax.ShapeDtypeStruct((NBLK, Q, GB), jnp.float32),
        ],
        compiler_params=pltpu.CompilerParams(
            dimension_semantics=("parallel", "parallel"),
        ),
    )(x, k_sq, q_sq, keys_p)

    gmin = jnp.transpose(gmin3, (1, 0, 2)).reshape(Q, NG)

    ridx, kbase = pl.pallas_call(
        _stage2,
        grid=(Q // QB2,),
        in_specs=[pl.BlockSpec((QB2, NG), lambda j: (j, 0))],
        out_specs=[
            pl.BlockSpec((QB2, TOPK), lambda j: (j, 0)),
            pl.BlockSpec((QB2, TOPK), lambda j: (j, 0)),
        ],
        out_shape=[
            jax.ShapeDtypeStruct((Q, TOPK), jnp.int32),
            jax.ShapeDtypeStruct((Q, TOPK), jnp.int32),
        ],
        compiler_params=pltpu.CompilerParams(
            dimension_semantics=("arbitrary",),
        ),
    )(gmin)

    stage3 = functools.partial(
        pl.kernel,
        out_type=[
            jax.ShapeDtypeStruct((Q, 16), jnp.float32),
            jax.ShapeDtypeStruct((Q, 16), jnp.int32),
        ],
        mesh=plsc.VectorSubcoreMesh(core_axis_name="c", subcore_axis_name="s"),
        scratch_types=[
            pltpu.VMEM((NR,), jnp.int32),
            pltpu.VMEM((NR,), jnp.int32),
            pltpu.VMEM((NR, GSZ), jnp.float32),
            pltpu.VMEM((QW, 16), jnp.float32),
            pltpu.VMEM((QW, 16), jnp.int32),
            pltpu.SemaphoreType.DMA,
        ],
    )(_stage3)
    dpad, ipad = stage3(dists3.reshape(R1, GSZ),
                        ridx.reshape(Q * TOPK),
                        kbase.reshape(Q * TOPK))
    return (dpad[:, :TOPK], ipad[:, :TOPK])
